# baseline (device time: 746489 ns/iter reference)
import functools

import jax
import jax.numpy as jnp
from jax import lax
from jax.experimental import pallas as pl
from jax.experimental.pallas import tpu as pltpu

N_DEV = 16


def kernel(x, w_mat, scale_x, scale_w):
    m_global, k_per = x.shape
    _, n = w_mat.shape
    m_per = m_global // N_DEV

    x_bf = x.astype(jnp.bfloat16)
    w_bf = w_mat.astype(jnp.bfloat16)

    def body(x_ref, w_ref, sx_ref, sw_ref, out_ref,
             send_buf, recv_buf, send_sems, recv_sems, credit_sem):
        my = lax.axis_index("i")
        left = lax.rem(my + N_DEV - 1, N_DEV)
        right = lax.rem(my + 1, N_DEV)

        barrier_sem = pltpu.get_barrier_semaphore()
        for nbr in (left, right):
            pl.semaphore_signal(barrier_sem, inc=1, device_id=(nbr,),
                                device_id_type=pl.DeviceIdType.MESH)
        pl.semaphore_wait(barrier_sem, 2)

        alpha = sx_ref[0] * sw_ref[0]

        rdmas = [None] * N_DEV
        for h in range(N_DEV):
            c = lax.rem(my + h + 1, N_DEV)
            part = jnp.dot(
                x_ref[pl.ds(c * m_per, m_per), :],
                w_ref[...],
                preferred_element_type=jnp.float32,
            )
            if h > 0:
                rdmas[h - 1].wait_recv()
                total = part + recv_buf[(h - 1) % 2, :, :].astype(jnp.float32)
                if h <= N_DEV - 3:
                    pl.semaphore_signal(credit_sem, inc=1, device_id=(right,),
                                        device_id_type=pl.DeviceIdType.MESH)
            else:
                total = part
            if h < N_DEV - 1:
                send_buf[h % 2, :, :] = total.astype(jnp.bfloat16)
                if h >= 2:
                    pl.semaphore_wait(credit_sem, 1)
                rdma = pltpu.make_async_remote_copy(
                    src_ref=send_buf.at[h % 2],
                    dst_ref=recv_buf.at[h % 2],
                    send_sem=send_sems.at[h % 2],
                    recv_sem=recv_sems.at[h % 2],
                    device_id=(left,),
                    device_id_type=pl.DeviceIdType.MESH,
                )
                rdma.start()
                rdma.wait_send()
                rdmas[h] = rdma
            else:
                out_ref[...] = jnp.maximum(total * alpha, 0.0)

        @functools.partial(pl.run_scoped, sem2=pltpu.SemaphoreType.REGULAR)
        def _(sem2):
            for nbr in (left, right):
                pl.semaphore_signal(sem2, inc=1, device_id=(nbr,),
                                    device_id_type=pl.DeviceIdType.MESH)
            pl.semaphore_wait(sem2, 2)

    return pl.pallas_call(
        body,
        out_shape=jax.ShapeDtypeStruct((m_per, n), jnp.float32),
        in_specs=[
            pl.BlockSpec(memory_space=pltpu.VMEM),
            pl.BlockSpec(memory_space=pltpu.VMEM),
            pl.BlockSpec(memory_space=pltpu.SMEM),
            pl.BlockSpec(memory_space=pltpu.SMEM),
        ],
        out_specs=pl.BlockSpec(memory_space=pltpu.VMEM),
        scratch_shapes=[
            pltpu.VMEM((2, m_per, n), jnp.bfloat16),
            pltpu.VMEM((2, m_per, n), jnp.bfloat16),
            pltpu.SemaphoreType.DMA((2,)),
            pltpu.SemaphoreType.DMA((2,)),
            pltpu.SemaphoreType.REGULAR,
        ],
        compiler_params=pltpu.CompilerParams(collective_id=0),
    )(x_bf, w_bf, scale_x, scale_w)


# device time: 426842 ns/iter; 1.7489x vs baseline; 1.7489x over previous
import functools

import jax
import jax.numpy as jnp
from jax import lax
from jax.experimental import pallas as pl
from jax.experimental.pallas import tpu as pltpu

N_DEV = 16


def kernel(x, w_mat, scale_x, scale_w):
    m_global, k_per = x.shape
    _, n = w_mat.shape
    m_per = m_global // N_DEV
    half = n // 2

    x_bf = x.astype(jnp.bfloat16)
    w_bf = w_mat.astype(jnp.bfloat16)

    def body(x_ref, w_ref, sx_ref, sw_ref, out_ref,
             send_l, recv_l, send_r, recv_r,
             ssem_l, rsem_l, ssem_r, rsem_r, credit_l, credit_r):
        my = lax.axis_index("i")
        left = lax.rem(my + N_DEV - 1, N_DEV)
        right = lax.rem(my + 1, N_DEV)

        barrier_sem = pltpu.get_barrier_semaphore()
        for nbr in (left, right):
            pl.semaphore_signal(barrier_sem, inc=1, device_id=(nbr,),
                                device_id_type=pl.DeviceIdType.MESH)
        pl.semaphore_wait(barrier_sem, 2)

        alpha = sx_ref[0] * sw_ref[0]

        rdmas_l = [None] * N_DEV
        rdmas_r = [None] * N_DEV
        for h in range(N_DEV):
            cl = lax.rem(my + h + 1, N_DEV)
            cr = lax.rem(my + 2 * N_DEV - h - 1, N_DEV)
            part_l = jnp.dot(
                x_ref[pl.ds(cl * m_per, m_per), :],
                w_ref[:, :half],
                preferred_element_type=jnp.float32,
            )
            part_r = jnp.dot(
                x_ref[pl.ds(cr * m_per, m_per), :],
                w_ref[:, half:],
                preferred_element_type=jnp.float32,
            )
            if h > 0:
                rdmas_l[h - 1].wait_recv()
                total_l = part_l + recv_l[(h - 1) % 2, :, :].astype(jnp.float32)
                rdmas_r[h - 1].wait_recv()
                total_r = part_r + recv_r[(h - 1) % 2, :, :].astype(jnp.float32)
                if h <= N_DEV - 3:
                    pl.semaphore_signal(credit_l, inc=1, device_id=(right,),
                                        device_id_type=pl.DeviceIdType.MESH)
                    pl.semaphore_signal(credit_r, inc=1, device_id=(left,),
                                        device_id_type=pl.DeviceIdType.MESH)
            else:
                total_l = part_l
                total_r = part_r
            if h < N_DEV - 1:
                if h >= 2:
                    rdmas_l[h - 2].wait_send()
                    rdmas_r[h - 2].wait_send()
                send_l[h % 2, :, :] = total_l.astype(jnp.bfloat16)
                send_r[h % 2, :, :] = total_r.astype(jnp.bfloat16)
                if h >= 2:
                    pl.semaphore_wait(credit_l, 1)
                    pl.semaphore_wait(credit_r, 1)
                rdma_l = pltpu.make_async_remote_copy(
                    src_ref=send_l.at[h % 2],
                    dst_ref=recv_l.at[h % 2],
                    send_sem=ssem_l.at[h % 2],
                    recv_sem=rsem_l.at[h % 2],
                    device_id=(left,),
                    device_id_type=pl.DeviceIdType.MESH,
                )
                rdma_r = pltpu.make_async_remote_copy(
                    src_ref=send_r.at[h % 2],
                    dst_ref=recv_r.at[h % 2],
                    send_sem=ssem_r.at[h % 2],
                    recv_sem=rsem_r.at[h % 2],
                    device_id=(right,),
                    device_id_type=pl.DeviceIdType.MESH,
                )
                rdma_l.start()
                rdma_r.start()
                rdmas_l[h] = rdma_l
                rdmas_r[h] = rdma_r
            else:
                out_ref[:, :half] = jnp.maximum(total_l * alpha, 0.0)
                out_ref[:, half:] = jnp.maximum(total_r * alpha, 0.0)

        for h in (N_DEV - 3, N_DEV - 2):
            rdmas_l[h].wait_send()
            rdmas_r[h].wait_send()

        @functools.partial(pl.run_scoped, sem2=pltpu.SemaphoreType.REGULAR)
        def _(sem2):
            for nbr in (left, right):
                pl.semaphore_signal(sem2, inc=1, device_id=(nbr,),
                                    device_id_type=pl.DeviceIdType.MESH)
            pl.semaphore_wait(sem2, 2)

    return pl.pallas_call(
        body,
        out_shape=jax.ShapeDtypeStruct((m_per, n), jnp.float32),
        in_specs=[
            pl.BlockSpec(memory_space=pltpu.VMEM),
            pl.BlockSpec(memory_space=pltpu.VMEM),
            pl.BlockSpec(memory_space=pltpu.SMEM),
            pl.BlockSpec(memory_space=pltpu.SMEM),
        ],
        out_specs=pl.BlockSpec(memory_space=pltpu.VMEM),
        scratch_shapes=[
            pltpu.VMEM((2, m_per, half), jnp.bfloat16),
            pltpu.VMEM((2, m_per, half), jnp.bfloat16),
            pltpu.VMEM((2, m_per, half), jnp.bfloat16),
            pltpu.VMEM((2, m_per, half), jnp.bfloat16),
            pltpu.SemaphoreType.DMA((2,)),
            pltpu.SemaphoreType.DMA((2,)),
            pltpu.SemaphoreType.DMA((2,)),
            pltpu.SemaphoreType.DMA((2,)),
            pltpu.SemaphoreType.REGULAR,
            pltpu.SemaphoreType.REGULAR,
        ],
        compiler_params=pltpu.CompilerParams(collective_id=0),
    )(x_bf, w_bf, scale_x, scale_w)


# device time: 358654 ns/iter; 2.0814x vs baseline; 1.1901x over previous
import functools

import jax
import jax.numpy as jnp
from jax import lax
from jax.experimental import pallas as pl
from jax.experimental.pallas import tpu as pltpu

N_DEV = 16
NSUB = 2


def kernel(x, w_mat, scale_x, scale_w):
    m_global, k_per = x.shape
    _, n = w_mat.shape
    m_per = m_global // N_DEV
    half = n // 2
    sub = half // NSUB

    x_bf = x.astype(jnp.bfloat16)
    w_bf = w_mat.astype(jnp.bfloat16)

    def body(x_ref, w_ref, sx_ref, sw_ref, out_ref,
             send_l, recv_l, send_r, recv_r,
             ssem_l, rsem_l, ssem_r, rsem_r, credit_l, credit_r):
        my = lax.axis_index("i")
        left = lax.rem(my + N_DEV - 1, N_DEV)
        right = lax.rem(my + 1, N_DEV)

        barrier_sem = pltpu.get_barrier_semaphore()
        for nbr in (left, right):
            pl.semaphore_signal(barrier_sem, inc=1, device_id=(nbr,),
                                device_id_type=pl.DeviceIdType.MESH)
        pl.semaphore_wait(barrier_sem, 2)

        alpha = sx_ref[0] * sw_ref[0]

        def make_rdma(send_buf, recv_buf, ssem, rsem, slot, s, nbr):
            k = slot * NSUB + s
            return pltpu.make_async_remote_copy(
                src_ref=send_buf.at[k],
                dst_ref=recv_buf.at[k],
                send_sem=ssem.at[k],
                recv_sem=rsem.at[k],
                device_id=(nbr,),
                device_id_type=pl.DeviceIdType.MESH,
            )

        rdmas_l = [[None] * NSUB for _ in range(N_DEV)]
        rdmas_r = [[None] * NSUB for _ in range(N_DEV)]
        for h in range(N_DEV):
            slot = h % 2
            cl = lax.rem(my + h + 1, N_DEV)
            cr = lax.rem(my + 2 * N_DEV - h - 1, N_DEV)
            part_l = jnp.dot(x_ref[pl.ds(cl * m_per, m_per), :],
                             w_ref[:, :half],
                             preferred_element_type=jnp.float32)
            part_r = jnp.dot(x_ref[pl.ds(cr * m_per, m_per), :],
                             w_ref[:, half:],
                             preferred_element_type=jnp.float32)

            if 2 <= h < N_DEV - 1:
                for s in range(NSUB):
                    rdmas_l[h - 2][s].wait_send()
                    rdmas_r[h - 2][s].wait_send()
                pl.semaphore_wait(credit_l, 1)
                pl.semaphore_wait(credit_r, 1)

            for s in range(NSUB):
                cols = pl.ds(s * sub, sub)
                for (ring_rdmas, recv_buf, send_buf, ssem, rsem, part, nbr,
                     out_cols) in (
                        (rdmas_l, recv_l, send_l, ssem_l, rsem_l, part_l,
                         left, pl.ds(s * sub, sub)),
                        (rdmas_r, recv_r, send_r, ssem_r, rsem_r, part_r,
                         right, pl.ds(half + s * sub, sub))):
                    if h == 0:
                        total = part[:, s * sub:(s + 1) * sub]
                    else:
                        ring_rdmas[h - 1][s].wait_recv()
                        rec = recv_buf[((h - 1) % 2) * NSUB + s, :, :]
                        total = (part[:, s * sub:(s + 1) * sub]
                                 + rec.astype(jnp.float32))
                    if h < N_DEV - 1:
                        send_buf[slot * NSUB + s, :, :] = (
                            total.astype(jnp.bfloat16))
                        rdma = make_rdma(send_buf, recv_buf, ssem, rsem,
                                         slot, s, nbr)
                        rdma.start()
                        ring_rdmas[h][s] = rdma
                    else:
                        out_ref[:, out_cols] = jnp.maximum(total * alpha, 0.0)

            if 1 <= h <= N_DEV - 3:
                pl.semaphore_signal(credit_l, inc=1, device_id=(right,),
                                    device_id_type=pl.DeviceIdType.MESH)
                pl.semaphore_signal(credit_r, inc=1, device_id=(left,),
                                    device_id_type=pl.DeviceIdType.MESH)

        for h in (N_DEV - 3, N_DEV - 2):
            for s in range(NSUB):
                rdmas_l[h][s].wait_send()
                rdmas_r[h][s].wait_send()

        @functools.partial(pl.run_scoped, sem2=pltpu.SemaphoreType.REGULAR)
        def _(sem2):
            for nbr in (left, right):
                pl.semaphore_signal(sem2, inc=1, device_id=(nbr,),
                                    device_id_type=pl.DeviceIdType.MESH)
            pl.semaphore_wait(sem2, 2)

    return pl.pallas_call(
        body,
        out_shape=jax.ShapeDtypeStruct((m_per, n), jnp.float32),
        in_specs=[
            pl.BlockSpec(memory_space=pltpu.VMEM),
            pl.BlockSpec(memory_space=pltpu.VMEM),
            pl.BlockSpec(memory_space=pltpu.SMEM),
            pl.BlockSpec(memory_space=pltpu.SMEM),
        ],
        out_specs=pl.BlockSpec(memory_space=pltpu.VMEM),
        scratch_shapes=[
            pltpu.VMEM((2 * NSUB, m_per, sub), jnp.bfloat16),
            pltpu.VMEM((2 * NSUB, m_per, sub), jnp.bfloat16),
            pltpu.VMEM((2 * NSUB, m_per, sub), jnp.bfloat16),
            pltpu.VMEM((2 * NSUB, m_per, sub), jnp.bfloat16),
            pltpu.SemaphoreType.DMA((2 * NSUB,)),
            pltpu.SemaphoreType.DMA((2 * NSUB,)),
            pltpu.SemaphoreType.DMA((2 * NSUB,)),
            pltpu.SemaphoreType.DMA((2 * NSUB,)),
            pltpu.SemaphoreType.REGULAR,
            pltpu.SemaphoreType.REGULAR,
        ],
        compiler_params=pltpu.CompilerParams(collective_id=0),
    )(x_bf, w_bf, scale_x, scale_w)
